# Initial kernel scaffold; baseline (speedup 1.0000x reference)
import jax
import jax.numpy as jnp
from jax.experimental import pallas as pl


def kernel(x, edge_index, batch, W, att_src, att_dst, conv_b, gamma, beta, fcW, fcb):
    raise NotImplementedError("write your pallas kernel here")


# speedup vs baseline: 22.2860x; 22.2860x over previous
import functools

import jax
import jax.numpy as jnp
from jax import lax
from jax.experimental import pallas as pl
from jax.experimental.pallas import tpu as pltpu
from jax.experimental.pallas import tpu_sc as plsc

_D = 128
_CHUNK = 128
_NW = 32


def _mm_body(x_ref, w_ref, asr_ref, adr_ref, hm_ref, aa_ref):
    hm = jnp.dot(x_ref[...], w_ref[...], preferred_element_type=jnp.float32)
    hm_ref[...] = hm
    aa_ref[0, :] = jnp.sum(hm * asr_ref[...], axis=1)
    aa_ref[1, :] = jnp.sum(hm * adr_ref[...], axis=1)


def _matmul_attn(h, Wi, asr, adr):
    n = h.shape[0]
    return pl.pallas_call(
        _mm_body,
        out_shape=[
            jax.ShapeDtypeStruct((n, _D), jnp.float32),
            jax.ShapeDtypeStruct((2, n), jnp.float32),
        ],
    )(h, Wi, asr, adr)


def _post_body(acc_ref, den_ref, b_ref, hp_ref, st_ref):
    n = hp_ref.shape[0]
    npad = acc_ref.shape[0] // 2
    num = acc_ref[0:n, 0:_D] + acc_ref[npad:npad + n, 0:_D]
    ones = jnp.ones((den_ref.shape[0], 1), jnp.float32)
    den = lax.dot_general(den_ref[...], ones, (((0,), (0,)), ((), ())),
                          preferred_element_type=jnp.float32)
    hp = num / (den[0:n] + 1e-16) + b_ref[...]
    hp_ref[...] = hp
    st_ref[0:1, :] = jnp.sum(hp, axis=0, keepdims=True)
    st_ref[1:2, :] = jnp.sum(hp * hp, axis=0, keepdims=True)


def _post(acc, den, bi, n):
    return pl.pallas_call(
        _post_body,
        out_shape=[
            jax.ShapeDtypeStruct((n, _D), jnp.float32),
            jax.ShapeDtypeStruct((2, _D), jnp.float32),
        ],
    )(acc, den, bi)


def _norm_body(hp_ref, st_ref, g_ref, be_ref, out_ref):
    n = hp_ref.shape[0]
    mean = st_ref[0:1, :] / n
    var = st_ref[1:2, :] / n - mean * mean
    h = (hp_ref[...] - mean) / jnp.sqrt(var + 1e-5) * g_ref[...] + be_ref[...]
    out_ref[...] = jnp.maximum(h, 0.0)


def _norm(hp, st, gi, bei):
    n = hp.shape[0]
    return pl.pallas_call(
        _norm_body,
        out_shape=jax.ShapeDtypeStruct((n, _D), jnp.float32),
    )(hp, st, gi, bei)


def _final_body(h_ref, b_ref, w_ref, fb_ref, out_ref, pooled_ref):
    i = pl.program_id(0)
    nb = pl.num_programs(0)
    blk = h_ref.shape[0]

    @pl.when(i == 0)
    def _():
        pooled_ref[...] = jnp.zeros_like(pooled_ref)

    bblk = b_ref[0, 0, :]
    gi = lax.broadcasted_iota(jnp.int32, (blk, 64), 1)
    onehot = (bblk[:, None] == gi).astype(jnp.float32)
    pooled_ref[...] += lax.dot_general(
        onehot, h_ref[...], (((0,), (0,)), ((), ())),
        preferred_element_type=jnp.float32)

    @pl.when(i == nb - 1)
    def _():
        out = jnp.dot(pooled_ref[...], w_ref[...],
                      preferred_element_type=jnp.float32) + fb_ref[...]
        m = jnp.max(out, axis=1, keepdims=True)
        z = out - m
        lse = jnp.log(jnp.sum(jnp.exp(z), axis=1, keepdims=True))
        out_ref[...] = z - lse


def _final(Hcat, batch2, fcWr, fcbs):
    n, dcat = Hcat.shape
    blk = n // 10
    return pl.pallas_call(
        _final_body,
        grid=(10,),
        in_specs=[
            pl.BlockSpec((blk, dcat), lambda i: (i, 0)),
            pl.BlockSpec((1, 1, blk), lambda i: (i, 0, 0)),
            pl.BlockSpec((dcat, 64), lambda i: (0, 0)),
            pl.BlockSpec((1, 64), lambda i: (0, 0)),
        ],
        out_specs=pl.BlockSpec((64, 64), lambda i: (0, 0)),
        out_shape=jax.ShapeDtypeStruct((64, 64), jnp.float32),
        scratch_shapes=[pltpu.VMEM((64, dcat), jnp.float32)],
    )(Hcat, batch2, fcWr, fcbs)


def _make_sc_edge(n, etot, epad):
    epw = epad // _NW
    nch = epw // _CHUNK
    mesh = plsc.VectorSubcoreMesh(core_axis_name="c", subcore_axis_name="s")
    slab = ((n + 16 * 8 - 1) // (16 * 8)) * 8
    npad = 16 * slab

    @functools.partial(
        pl.kernel, mesh=mesh,
        compiler_params=pltpu.CompilerParams(needs_layout_passes=False),
        out_type=[
            jax.ShapeDtypeStruct((2 * npad, _D), jnp.float32),
            jax.ShapeDtypeStruct((_NW * n,), jnp.float32),
        ],
        scratch_types=[
            pltpu.VMEM((_CHUNK,), jnp.int32),
            pltpu.VMEM((_CHUNK,), jnp.int32),
            pltpu.VMEM((n,), jnp.float32),
            pltpu.VMEM((n,), jnp.float32),
            pltpu.VMEM((n,), jnp.float32),
            pltpu.VMEM((_CHUNK,), jnp.float32),
            pltpu.VMEM((_CHUNK, _D), jnp.float32),
            pltpu.VMEM_SHARED((npad, _D), jnp.float32),
            pltpu.SemaphoreType.DMA,
        ],
    )
    def sc_edge(h_hbm, aa_hbm, src_hbm, dst_hbm, num_hbm, den_hbm,
                srcv, dstv, asv, adv, denv, exv, rowsv, acc, sem):
        cid = lax.axis_index("c")
        sid = lax.axis_index("s")
        wid = cid * 16 + sid
        lanes = lax.iota(jnp.int32, 16)

        pltpu.sync_copy(aa_hbm.at[0], asv)
        pltpu.sync_copy(aa_hbm.at[1], adv)

        def zden(i, _):
            denv[pl.ds(i * 16, 16)] = jnp.zeros((16,), jnp.float32)
            return 0
        lax.fori_loop(0, n // 16, zden, 0)

        def zrow(i, _):
            for j in range(_D // 16):
                rowsv[i, pl.ds(j * 16, 16)] = jnp.zeros((16,), jnp.float32)
            return 0
        lax.fori_loop(0, _CHUNK, zrow, 0)
        off = 0
        while off < slab:
            zr = min(_CHUNK, slab - off)
            pltpu.sync_copy(rowsv.at[pl.ds(0, zr)],
                            acc.at[pl.ds(sid * slab + off, zr)])
            off += zr
        plsc.subcore_barrier()

        def chunk(c, _):
            base = wid * epw + c * _CHUNK
            pltpu.sync_copy(src_hbm.at[pl.ds(base, _CHUNK)], srcv)
            pltpu.sync_copy(dst_hbm.at[pl.ds(base, _CHUNK)], dstv)
            for j in range(_CHUNK // 16):
                s16 = srcv[pl.ds(j * 16, 16)]
                d16 = dstv[pl.ds(j * 16, 16)]
                e = plsc.load_gather(asv, [s16]) + plsc.load_gather(adv, [d16])
                e = jnp.where(e > 0, e, 0.2 * e)
                ex = jnp.exp(e)
                gid = base + j * 16 + lanes
                ex = jnp.where(gid < etot, ex, 0.0)
                exv[pl.ds(j * 16, 16)] = ex
                dk, ev = plsc.sort_key_val(d16, ex)
                for s in (1, 2, 4, 8):
                    sidx = jnp.maximum(lanes - s, 0)
                    kk = dk.at[sidx].get(mode="promise_in_bounds")
                    vv = ev.at[sidx].get(mode="promise_in_bounds")
                    ev = ev + jnp.where((kk == dk) & (lanes >= s), vv, 0.0)
                nxt = dk.at[jnp.minimum(lanes + 1, 15)].get(
                    mode="promise_in_bounds")
                is_end = (lanes == 15) | (nxt != dk)
                g = plsc.load_gather(denv, [dk])
                plsc.store_scatter(denv, [dk], g + ev, mask=is_end)

            pltpu.async_copy(h_hbm.at[srcv], rowsv, sem).wait()

            def gbody(g2, _):
                ex16 = exv[pl.ds(g2 * 16, 16)]
                for lane in range(16):
                    i = g2 * 16 + lane
                    exs = ex16[lane]
                    for j in range(_D // 16):
                        rowsv[i, pl.ds(j * 16, 16)] = (
                            rowsv[i, pl.ds(j * 16, 16)] * exs)
                return 0
            lax.fori_loop(0, _CHUNK // 16, gbody, 0)

            pltpu.sync_copy(rowsv, acc.at[dstv], add=True)
            return 0

        lax.fori_loop(0, nch, chunk, 0)
        plsc.subcore_barrier()

        pltpu.sync_copy(
            acc.at[pl.ds(sid * slab, slab)],
            num_hbm.at[pl.ds(cid * npad + sid * slab, slab)])
        pltpu.sync_copy(denv, den_hbm.at[pl.ds(wid * n, n)])

    return sc_edge


def kernel(x, edge_index, batch, W, att_src, att_dst, conv_b, gamma, beta,
           fcW, fcb):
    n, d = x.shape
    nl = W.shape[0]
    loop = jnp.arange(n, dtype=edge_index.dtype)
    src = jnp.concatenate([edge_index[0], loop])
    dst = jnp.concatenate([edge_index[1], loop])
    etot = src.shape[0]
    epad = ((etot + _NW * _CHUNK - 1) // (_NW * _CHUNK)) * (_NW * _CHUNK)
    srcp = jnp.pad(src, (0, epad - etot))
    dstp = jnp.pad(dst, (0, epad - etot))

    sc_edge = _make_sc_edge(n, etot, epad)

    h = x
    outs = [x]
    for i in range(nl):
        hm, aa = _matmul_attn(h, W[i], att_src[i].reshape(1, d),
                              att_dst[i].reshape(1, d))
        acc, den = sc_edge(hm, aa, srcp, dstp)
        hp, st = _post(acc, den.reshape(_NW, n), conv_b[i].reshape(1, d), n)
        h = _norm(hp, st, gamma[i].reshape(1, d), beta[i].reshape(1, d))
        outs.append(h)

    Hcat = jnp.concatenate(outs, axis=1)
    fcWr = fcW.reshape((nl + 1) * d, fcW.shape[2])
    fcbs = fcb.sum(axis=0).reshape(1, fcb.shape[1])
    return _final(Hcat, batch.reshape(10, 1, n // 10), fcWr, fcbs)
